# baseline (speedup 1.0000x reference)
import jax
import jax.numpy as jnp
from jax.experimental import pallas as pl
from jax.experimental.pallas import tpu as pltpu
from jax.experimental.pallas import tpu_sc as plsc

M = 16384
D = 256
K = 8192

BM = 2048
BN = 1024
LANES = 128
JC = BN // LANES
RG = 256
GW = 128


def _vq_body(x_ref, w_ref, b_ref, cb_ref, esq_ref,
             idx_ref, loss_ref, min_ref, arg_ref):
    z = jax.lax.dot_general(
        x_ref[...], w_ref[...], (((1,), (1,)), ((), ())),
        preferred_element_type=jnp.float32)
    z = z + b_ref[...]
    zsq = jnp.sum(z * z, axis=1, keepdims=True)
    z2 = z + z

    min_ref[...] = jnp.full((BM, LANES), jnp.inf, jnp.float32)
    arg_ref[...] = jnp.zeros((BM, LANES), jnp.int32)

    for c in range(K // BN):
        dot2 = jax.lax.dot_general(
            z2, cb_ref[c * BN:(c + 1) * BN, :],
            (((1,), (1,)), ((), ())),
            preferred_element_type=jnp.float32)
        for r in range(BM // RG):
            rs = slice(r * RG, (r + 1) * RG)
            accm = min_ref[rs, :]
            acci = arg_ref[rs, :]
            zsq_r = zsq[r * RG:(r + 1) * RG, :]
            for j in range(JC):
                o = c * JC + j
                dj = (zsq_r - dot2[r * RG:(r + 1) * RG,
                                   j * LANES:(j + 1) * LANES]) \
                    + esq_ref[0:1, o * LANES:(o + 1) * LANES]
                upd = dj < accm
                accm = jnp.where(upd, dj, accm)
                acci = jnp.where(upd, o, acci)
            min_ref[rs, :] = accm
            arg_ref[rs, :] = acci

    accm = min_ref[...]
    gidx = arg_ref[...] * LANES \
        + jax.lax.broadcasted_iota(jnp.int32, (BM, LANES), 1)
    rowmin = jnp.min(accm, axis=1, keepdims=True)
    cand = jnp.where(accm == rowmin, gidx, jnp.int32(2147483647))
    idx_ref[...] = jnp.min(cand, axis=1).reshape(1, BM)

    @pl.when(pl.program_id(0) == 0)
    def _():
        loss_ref[...] = jnp.zeros((1, 1), jnp.float32)

    loss_ref[...] += jnp.sum(rowmin)[None, None]


def _vq_argmin(xf, W, b2, codebook, esq):
    return pl.pallas_call(
        _vq_body,
        grid=(M // BM,),
        in_specs=[
            pl.BlockSpec((BM, D), lambda m: (m, 0)),
            pl.BlockSpec((D, D), lambda m: (0, 0)),
            pl.BlockSpec((1, D), lambda m: (0, 0)),
            pl.BlockSpec((K, D), lambda m: (0, 0)),
            pl.BlockSpec((1, K), lambda m: (0, 0)),
        ],
        out_specs=[
            pl.BlockSpec((1, BM), lambda m: (0, m)),
            pl.BlockSpec((1, 1), lambda m: (0, 0)),
        ],
        out_shape=[
            jax.ShapeDtypeStruct((1, M), jnp.int32),
            jax.ShapeDtypeStruct((1, 1), jnp.float32),
        ],
        scratch_shapes=[
            pltpu.VMEM((BM, LANES), jnp.float32),
            pltpu.VMEM((BM, LANES), jnp.int32),
        ],
        compiler_params=pltpu.CompilerParams(
            dimension_semantics=("arbitrary",)),
    )(xf, W, b2, codebook, esq)


def _sc_gather(codebook, idx_row):
    mesh = plsc.VectorSubcoreMesh(core_axis_name="c", subcore_axis_name="s")

    @pl.kernel(out_type=jax.ShapeDtypeStruct((M, D), jnp.float32), mesh=mesh)
    def gather_kernel(cb_hbm, i_hbm, o_hbm):
        def body(i_vmem, o_vmem):
            pltpu.sync_copy(cb_hbm.at[i_vmem.at[0]], o_vmem)

        pltpu.emit_pipeline(
            body,
            grid=(M // GW,),
            in_specs=[pl.BlockSpec((1, GW), index_map=lambda i: (0, i))],
            out_specs=[pl.BlockSpec((GW, D), index_map=lambda i: (i, 0))],
            core_axis_name=("c", "s"),
            dimension_semantics=(pltpu.PARALLEL,),
        )(i_hbm, o_hbm)

    return gather_kernel(codebook, idx_row)


def kernel(x, W, b, codebook):
    xf = x.reshape(M, D)
    b2 = b.reshape(1, D)
    esq = jnp.sum(codebook ** 2, axis=1)[None, :]
    idx, loss_sum = _vq_argmin(xf, W, b2, codebook, esq)
    quantized = _sc_gather(codebook, idx)
    loss = loss_sum[0, 0] / jnp.float32(M * D)
    x_recon = quantized.reshape(x.shape)
    return loss, loss, x_recon


# speedup vs baseline: 1.0033x; 1.0033x over previous
import jax
import jax.numpy as jnp
from jax.experimental import pallas as pl
from jax.experimental.pallas import tpu as pltpu
from jax.experimental.pallas import tpu_sc as plsc

M = 16384
D = 256
K = 8192

BM = 2048
BN = 512
LANES = 128
JC = BN // LANES
RG = 256
GW = 128


def _vq_body(x_ref, w_ref, b_ref, cb_ref, esq_ref,
             idx_ref, loss_ref, min_ref, arg_ref):
    z = jax.lax.dot_general(
        x_ref[...], w_ref[...], (((1,), (1,)), ((), ())),
        preferred_element_type=jnp.float32)
    z = z + b_ref[...]
    zsq = jnp.sum(z * z, axis=1, keepdims=True)
    z2 = z + z

    min_ref[...] = jnp.full((BM, LANES), jnp.inf, jnp.float32)
    arg_ref[...] = jnp.zeros((BM, LANES), jnp.int32)

    for c in range(K // BN):
        dot2 = jax.lax.dot_general(
            z2, cb_ref[c * BN:(c + 1) * BN, :],
            (((1,), (1,)), ((), ())),
            preferred_element_type=jnp.float32)
        for r in range(BM // RG):
            rs = slice(r * RG, (r + 1) * RG)
            accm = min_ref[rs, :]
            acci = arg_ref[rs, :]
            zsq_r = zsq[r * RG:(r + 1) * RG, :]
            for j in range(JC):
                o = c * JC + j
                dj = (zsq_r - dot2[r * RG:(r + 1) * RG,
                                   j * LANES:(j + 1) * LANES]) \
                    + esq_ref[0:1, o * LANES:(o + 1) * LANES]
                upd = dj < accm
                accm = jnp.where(upd, dj, accm)
                acci = jnp.where(upd, o, acci)
            min_ref[rs, :] = accm
            arg_ref[rs, :] = acci

    accm = min_ref[...]
    gidx = arg_ref[...] * LANES \
        + jax.lax.broadcasted_iota(jnp.int32, (BM, LANES), 1)
    rowmin = jnp.min(accm, axis=1, keepdims=True)
    cand = jnp.where(accm == rowmin, gidx, jnp.int32(2147483647))
    idx_ref[...] = jnp.min(cand, axis=1).reshape(1, BM)

    @pl.when(pl.program_id(0) == 0)
    def _():
        loss_ref[...] = jnp.zeros((1, 1), jnp.float32)

    loss_ref[...] += jnp.sum(rowmin)[None, None]


def _vq_argmin(xf, W, b2, codebook, esq):
    return pl.pallas_call(
        _vq_body,
        grid=(M // BM,),
        in_specs=[
            pl.BlockSpec((BM, D), lambda m: (m, 0)),
            pl.BlockSpec((D, D), lambda m: (0, 0)),
            pl.BlockSpec((1, D), lambda m: (0, 0)),
            pl.BlockSpec((K, D), lambda m: (0, 0)),
            pl.BlockSpec((1, K), lambda m: (0, 0)),
        ],
        out_specs=[
            pl.BlockSpec((1, BM), lambda m: (0, m)),
            pl.BlockSpec((1, 1), lambda m: (0, 0)),
        ],
        out_shape=[
            jax.ShapeDtypeStruct((1, M), jnp.int32),
            jax.ShapeDtypeStruct((1, 1), jnp.float32),
        ],
        scratch_shapes=[
            pltpu.VMEM((BM, LANES), jnp.float32),
            pltpu.VMEM((BM, LANES), jnp.int32),
        ],
        compiler_params=pltpu.CompilerParams(
            dimension_semantics=("arbitrary",)),
    )(xf, W, b2, codebook, esq)


def _sc_gather(codebook, idx_row):
    mesh = plsc.VectorSubcoreMesh(core_axis_name="c", subcore_axis_name="s")

    @pl.kernel(out_type=jax.ShapeDtypeStruct((M, D), jnp.float32), mesh=mesh)
    def gather_kernel(cb_hbm, i_hbm, o_hbm):
        def body(i_vmem, o_vmem):
            pltpu.sync_copy(cb_hbm.at[i_vmem.at[0]], o_vmem)

        pltpu.emit_pipeline(
            body,
            grid=(M // GW,),
            in_specs=[pl.BlockSpec((1, GW), index_map=lambda i: (0, i))],
            out_specs=[pl.BlockSpec((GW, D), index_map=lambda i: (i, 0))],
            core_axis_name=("c", "s"),
            dimension_semantics=(pltpu.PARALLEL,),
        )(i_hbm, o_hbm)

    return gather_kernel(codebook, idx_row)


def kernel(x, W, b, codebook):
    xf = x.reshape(M, D)
    b2 = b.reshape(1, D)
    esq = jnp.sum(codebook ** 2, axis=1)[None, :]
    idx, loss_sum = _vq_argmin(xf, W, b2, codebook, esq)
    quantized = _sc_gather(codebook, idx)
    loss = loss_sum[0, 0] / jnp.float32(M * D)
    x_recon = quantized.reshape(x.shape)
    return loss, loss, x_recon


# speedup vs baseline: 1.0378x; 1.0343x over previous
import jax
import jax.numpy as jnp
from jax.experimental import pallas as pl
from jax.experimental.pallas import tpu as pltpu
from jax.experimental.pallas import tpu_sc as plsc

M = 16384
D = 256
K = 8192

BM = 2048
BN = 512
LANES = 128
JC = BN // LANES
RG = 256
GW = 128


def _vq_body(x_ref, w_ref, b_ref, cb_ref, esq_ref,
             idx_ref, loss_ref, min_ref, arg_ref):
    z = jax.lax.dot_general(
        x_ref[...], w_ref[...], (((1,), (1,)), ((), ())),
        preferred_element_type=jnp.float32)
    z = z + b_ref[...]
    zsq = jnp.sum(z * z, axis=1, keepdims=True)
    z2 = z + z

    min_ref[...] = jnp.full((BM, LANES), jnp.inf, jnp.float32)
    arg_ref[...] = jnp.zeros((BM, LANES), jnp.int32)

    for c in range(K // BN):
        dot2 = jax.lax.dot_general(
            z2, cb_ref[c * BN:(c + 1) * BN, :],
            (((1,), (1,)), ((), ())),
            preferred_element_type=jnp.float32)
        for r in range(BM // RG):
            rs = slice(r * RG, (r + 1) * RG)
            accm = min_ref[rs, :]
            acci = arg_ref[rs, :]
            zsq_r = zsq[rs, :]
            for j in range(JC):
                o = c * JC + j
                dj = (zsq_r - dot2[r * RG:(r + 1) * RG,
                                   j * LANES:(j + 1) * LANES]) \
                    + esq_ref[0:1, o * LANES:(o + 1) * LANES]
                upd = dj < accm
                accm = jnp.where(upd, dj, accm)
                acci = jnp.where(upd, o, acci)
            min_ref[rs, :] = accm
            arg_ref[rs, :] = acci

    accm = min_ref[...]
    gidx = arg_ref[...] * LANES \
        + jax.lax.broadcasted_iota(jnp.int32, (BM, LANES), 1)
    rowmin = jnp.min(accm, axis=1, keepdims=True)
    cand = jnp.where(accm == rowmin, gidx, jnp.int32(2147483647))
    idx_ref[...] = jnp.min(cand, axis=1, keepdims=True)

    @pl.when(pl.program_id(0) == 0)
    def _():
        loss_ref[...] = jnp.zeros((1, 1), jnp.float32)

    loss_ref[...] += jnp.sum(rowmin)[None, None]


def _vq_argmin(xf, W, b2, codebook, esq):
    return pl.pallas_call(
        _vq_body,
        grid=(M // BM,),
        in_specs=[
            pl.BlockSpec((BM, D), lambda m: (m, 0)),
            pl.BlockSpec((D, D), lambda m: (0, 0)),
            pl.BlockSpec((1, D), lambda m: (0, 0)),
            pl.BlockSpec((K, D), lambda m: (0, 0)),
            pl.BlockSpec((1, K), lambda m: (0, 0)),
        ],
        out_specs=[
            pl.BlockSpec((BM, 1), lambda m: (m, 0)),
            pl.BlockSpec((1, 1), lambda m: (0, 0)),
        ],
        out_shape=[
            jax.ShapeDtypeStruct((M, 1), jnp.int32),
            jax.ShapeDtypeStruct((1, 1), jnp.float32),
        ],
        scratch_shapes=[
            pltpu.VMEM((BM, LANES), jnp.float32),
            pltpu.VMEM((BM, LANES), jnp.int32),
        ],
        compiler_params=pltpu.CompilerParams(
            dimension_semantics=("arbitrary",)),
    )(xf, W, b2, codebook, esq)


def _sc_gather(codebook, idx_row):
    mesh = plsc.VectorSubcoreMesh(core_axis_name="c", subcore_axis_name="s")

    @pl.kernel(out_type=jax.ShapeDtypeStruct((M, D), jnp.float32), mesh=mesh)
    def gather_kernel(cb_hbm, i_hbm, o_hbm):
        def body(i_vmem, o_vmem):
            pltpu.sync_copy(cb_hbm.at[i_vmem.at[0]], o_vmem)

        pltpu.emit_pipeline(
            body,
            grid=(M // GW,),
            in_specs=[pl.BlockSpec((1, GW), index_map=lambda i: (0, i))],
            out_specs=[pl.BlockSpec((GW, D), index_map=lambda i: (i, 0))],
            core_axis_name=("c", "s"),
            dimension_semantics=(pltpu.PARALLEL,),
        )(i_hbm, o_hbm)

    return gather_kernel(codebook, idx_row)


def kernel(x, W, b, codebook):
    xf = x.reshape(M, D)
    b2 = b.reshape(1, D)
    esq = jnp.sum(codebook ** 2, axis=1)[None, :]
    idx, loss_sum = _vq_argmin(xf, W, b2, codebook, esq)
    quantized = _sc_gather(codebook, idx.reshape(1, M))
    loss = loss_sum[0, 0] / jnp.float32(M * D)
    x_recon = quantized.reshape(x.shape)
    return loss, loss, x_recon
